# baseline (speedup 1.0000x reference)
import functools

import jax
import jax.numpy as jnp
from jax import lax
from jax.experimental import pallas as pl
from jax.experimental.pallas import tpu as pltpu
from jax.experimental.pallas import tpu_sc as plsc

_D = 32
_NV = 1000000
_NB = 16384
_S = 50
_SP = 64
_NW = 32
_RB = _NB // _NW

_mesh = plsc.VectorSubcoreMesh(core_axis_name="c", subcore_axis_name="s")


@functools.partial(
    pl.kernel,
    out_type=jax.ShapeDtypeStruct((_NB * _SP,), jnp.int32),
    mesh=_mesh,
    scratch_types=[
        pltpu.VMEM((_RB, _S), jnp.int32),
        pltpu.VMEM((_RB * _SP,), jnp.int32),
    ],
)
def _prep(idx_hbm, idxp_hbm, vin, vout):
    wid = lax.axis_index("s") * 2 + lax.axis_index("c")
    i0 = wid * _RB
    pltpu.sync_copy(idx_hbm.at[pl.ds(i0, _RB)], vin)

    def row(r, carry):
        for k in (0, 16, 32, 34):
            vout[pl.ds(r * _SP + k, 16)] = vin[r, pl.ds(k, 16)]
        return carry

    lax.fori_loop(0, _RB, row, 0)
    pltpu.sync_copy(vout, idxp_hbm.at[pl.ds(i0 * _SP, _RB * _SP)])


_TC = 320
_TNCH = _NV // _TC
_TFULL = _TNCH // _NW
_TREM = _TNCH - _TFULL * _NW


@functools.partial(
    pl.kernel,
    out_type=jax.ShapeDtypeStruct((_NV * _D // 128, 128), jnp.float32),
    mesh=_mesh,
    scratch_types=[
        [pltpu.VMEM((_TC, _D), jnp.float32) for _ in range(2)],
        [pltpu.VMEM((_TC * _D // 128, 128), jnp.float32) for _ in range(2)],
        [pltpu.SemaphoreType.DMA for _ in range(2)],
        [pltpu.SemaphoreType.DMA for _ in range(2)],
    ],
)
def _tprep(tab_hbm, out_hbm, tin, tout, rsem, wsem):
    wid = lax.axis_index("s") * 2 + lax.axis_index("c")
    nch = _TFULL + jnp.where(wid < _TREM, 1, 0)

    def rd_start(c, b):
        pltpu.async_copy(tab_hbm.at[pl.ds(c * _TC, _TC)], tin[b], rsem[b])

    def rd_wait(c, b):
        pltpu.make_async_copy(
            tab_hbm.at[pl.ds(c * _TC, _TC)], tin[b], rsem[b]
        ).wait()

    def wr_start(c, b):
        pltpu.async_copy(
            tout[b], out_hbm.at[pl.ds(c * (_TC * _D // 128), _TC * _D // 128)],
            wsem[b],
        )

    def wr_wait(c, b):
        pltpu.make_async_copy(
            tout[b], out_hbm.at[pl.ds(c * (_TC * _D // 128), _TC * _D // 128)],
            wsem[b],
        ).wait()

    def repack(b):
        def line(l, carry):
            for j in range(4):
                for t in (0, 16):
                    tout[b][l, pl.ds(32 * j + t, 16)] = tin[b][
                        4 * l + j, pl.ds(t, 16)
                    ]
            return carry

        lax.fori_loop(0, _TC * _D // 128, line, 0)

    rd_start(wid, 0)
    rd_start(_NW + wid, 1)

    def pair_(p, carry):
        for b in range(2):
            k = 2 * p + b
            c = k * _NW + wid

            @pl.when(k < nch)
            def _():
                rd_wait(c, b)

                @pl.when(k >= 2)
                def _():
                    wr_wait(c - 2 * _NW, b)

                repack(b)
                wr_start(c, b)

                @pl.when(k + 2 < nch)
                def _():
                    rd_start(c + 2 * _NW, b)

        return carry

    lax.fori_loop(0, (_TFULL + 2) // 2, pair_, 0)

    @pl.when(nch == _TFULL)
    def _():
        wr_wait((_TFULL - 2) * _NW + wid, (_TFULL - 2) % 2)
        wr_wait((_TFULL - 1) * _NW + wid, (_TFULL - 1) % 2)

    @pl.when(nch == _TFULL + 1)
    def _():
        wr_wait((_TFULL - 1) * _NW + wid, (_TFULL - 1) % 2)
        wr_wait(_TFULL * _NW + wid, _TFULL % 2)


_NBUF = 4
_NGROUP = _RB // _NBUF


@functools.partial(
    pl.kernel,
    out_type=jax.ShapeDtypeStruct((_NB * _S, _D), jnp.float32),
    mesh=_mesh,
    scratch_types=[
        pltpu.VMEM((_RB * _SP,), jnp.int32),
        [pltpu.VMEM((_S, _D), jnp.float32) for _ in range(_NBUF)],
        [pltpu.SemaphoreType.DMA for _ in range(_NBUF)],
        [pltpu.SemaphoreType.DMA for _ in range(_NBUF)],
    ],
    compiler_params=pltpu.CompilerParams(use_tc_tiling_on_sc=False),
)
def _gather(idxp_hbm, table_hbm, out_hbm, idx_v, rows, gsem, wsem):
    wid = lax.axis_index("s") * 2 + lax.axis_index("c")
    i0 = wid * _RB
    pltpu.sync_copy(idxp_hbm.at[pl.ds(i0 * _SP, _RB * _SP)], idx_v)

    def gather_start(i, b):
        pltpu.async_copy(
            table_hbm.at[idx_v.at[pl.ds(i * _SP, _S)]], rows[b], gsem[b]
        )

    def gather_wait(i, b):
        pltpu.make_async_copy(
            table_hbm.at[idx_v.at[pl.ds(i * _SP, _S)]], rows[b], gsem[b]
        ).wait()

    def write_start(i, b):
        pltpu.async_copy(rows[b], out_hbm.at[pl.ds((i0 + i) * _S, _S)], wsem[b])

    def write_wait(i, b):
        pltpu.make_async_copy(
            rows[b], out_hbm.at[pl.ds((i0 + i) * _S, _S)], wsem[b]
        ).wait()

    for b in range(_NBUF):
        gather_start(b, b)

    def group(g, carry):
        i = g * _NBUF
        for b in range(_NBUF):
            gather_wait(i + b, b)
            write_start(i + b, b)

        @pl.when(g + 1 < _NGROUP)
        def _():
            for b in range(_NBUF):
                write_wait(i + b, b)
                gather_start(i + _NBUF + b, b)

        return carry

    lax.fori_loop(0, _NGROUP, group, 0)
    for b in range(_NBUF):
        write_wait(_RB - _NBUF + b, b)


_CH = 16
_NCH = _RB // _CH
_CHL = _CH * _S * _D // 128
_Q = 4
_QL = _Q * _S * _D // 128
_QR = _Q * _S
_NWQ = _RB // _Q


@functools.partial(
    pl.kernel,
    out_type=jax.ShapeDtypeStruct((_NB, _S, _D), jnp.float32),
    mesh=_mesh,
    scratch_types=[
        [pltpu.VMEM((_CHL, 128), jnp.float32) for _ in range(2)],
        [pltpu.VMEM((_QR, _D), jnp.float32) for _ in range(2)],
        [pltpu.SemaphoreType.DMA for _ in range(2)],
        [pltpu.SemaphoreType.DMA for _ in range(2)],
    ],
)
def _retile(flat_hbm, out_hbm, bufa, bufb, rsem, wsem):
    wid = lax.axis_index("s") * 2 + lax.axis_index("c")
    i0 = wid * _RB
    l0 = wid * (_RB * _S * _D // 128)

    def rd_start(ch, b):
        pltpu.async_copy(
            flat_hbm.at[pl.ds(l0 + ch * _CHL, _CHL)], bufa[b], rsem[b]
        )

    def rd_wait(ch, b):
        pltpu.make_async_copy(
            flat_hbm.at[pl.ds(l0 + ch * _CHL, _CHL)], bufa[b], rsem[b]
        ).wait()

    def wr_start(m, b):
        pltpu.async_copy(
            bufb[b].reshape(_Q, _S, _D),
            out_hbm.at[pl.ds(i0 + m * _Q, _Q)],
            wsem[b],
        )

    def wr_wait(m, b):
        pltpu.make_async_copy(
            bufb[b].reshape(_Q, _S, _D),
            out_hbm.at[pl.ds(i0 + m * _Q, _Q)],
            wsem[b],
        ).wait()

    def repack(ab, q, bb):
        def line(l, carry):
            for j in range(4):
                for t in (0, 16):
                    bufb[bb][4 * l + j, pl.ds(t, 16)] = bufa[ab][
                        q * _QL + l, pl.ds(32 * j + t, 16)
                    ]
            return carry

        lax.fori_loop(0, _QL, line, 0)

    for b in range(2):
        rd_start(b, b)

    def pair(p, carry):
        for b in range(2):
            ch = p * 2 + b
            rd_wait(ch, b)
            for q in range(4):
                m = ch * 4 + q
                qb = q % 2

                @pl.when(m >= 2)
                def _(m=m, qb=qb):
                    wr_wait(m - 2, qb)

                repack(b, q, qb)
                wr_start(m, qb)

            @pl.when(ch + 2 < _NCH)
            def _(b=b, ch=ch):
                rd_start(ch + 2, b)

        return carry

    lax.fori_loop(0, _NCH // 2, pair, 0)
    for b in range(2):
        wr_wait(_NWQ - 2 + b, b)


def kernel(input, embeddings):
    idxp = _prep(input.astype(jnp.int32))
    tab = _tprep(embeddings)
    flat = _gather(idxp, tab.reshape(_NV, _D))
    return _retile(flat.reshape(_NB * _S * _D // 128, 128))


# speedup vs baseline: 1.0766x; 1.0766x over previous
import functools

import jax
import jax.numpy as jnp
from jax import lax
from jax.experimental import pallas as pl
from jax.experimental.pallas import tpu as pltpu
from jax.experimental.pallas import tpu_sc as plsc

_D = 32
_NB = 16384
_S = 50
_NW = 32
_RB = _NB // _NW

_NBUF = 4
_NGROUP = _RB // _NBUF


_mesh = plsc.VectorSubcoreMesh(core_axis_name="c", subcore_axis_name="s")


@functools.partial(
    pl.kernel,
    out_type=jax.ShapeDtypeStruct((_NB, _S, _D), jnp.float32),
    mesh=_mesh,
    scratch_types=[
        pltpu.VMEM((_RB, _S), jnp.int32),
        [pltpu.VMEM((_S, _D), jnp.float32) for _ in range(_NBUF)],
        [pltpu.SemaphoreType.DMA for _ in range(_NBUF)],
        [pltpu.SemaphoreType.DMA for _ in range(_NBUF)],
    ],
    compiler_params=pltpu.CompilerParams(use_tc_tiling_on_sc=False),
)
def _gather_kernel(idx_hbm, table_hbm, out_hbm, idx_v, rows, gsem, wsem):
    wid = lax.axis_index("s") * 2 + lax.axis_index("c")
    base = wid * _RB
    pltpu.sync_copy(idx_hbm.at[pl.ds(base, _RB)], idx_v)

    def gather_start(i, b):
        pltpu.async_copy(table_hbm.at[idx_v.at[i]], rows[b], gsem[b])

    def gather_wait(i, b):
        pltpu.make_async_copy(table_hbm.at[idx_v.at[i]], rows[b], gsem[b]).wait()

    def write_start(i, b):
        pltpu.async_copy(rows[b], out_hbm.at[base + i], wsem[b])

    def write_wait(i, b):
        pltpu.make_async_copy(rows[b], out_hbm.at[base + i], wsem[b]).wait()

    for b in range(_NBUF):
        gather_start(b, b)

    def group(g, carry):
        i0 = g * _NBUF
        for b in range(_NBUF):
            gather_wait(i0 + b, b)
            write_start(i0 + b, b)
        @pl.when(g + 1 < _NGROUP)
        def _():
            for b in range(_NBUF):
                write_wait(i0 + b, b)
                gather_start(i0 + _NBUF + b, b)

        return carry

    lax.fori_loop(0, _NGROUP, group, 0)
    for b in range(_NBUF):
        write_wait(_RB - _NBUF + b, b)


def kernel(input, embeddings):
    return _gather_kernel(input.astype(jnp.int32), embeddings)


# speedup vs baseline: 1.1249x; 1.0448x over previous
import functools

import jax
import jax.numpy as jnp
from jax import lax
from jax.experimental import pallas as pl
from jax.experimental.pallas import tpu as pltpu
from jax.experimental.pallas import tpu_sc as plsc

_D = 32
_NB = 16384
_S = 50
_NW = 32
_RB = _NB // _NW
_P = _RB // 2

_NBUF = 4
_NGROUP = _P // _NBUF


_mesh = plsc.VectorSubcoreMesh(core_axis_name="c", subcore_axis_name="s")


@functools.partial(
    pl.kernel,
    out_type=jax.ShapeDtypeStruct((_NB, _S, _D), jnp.float32),
    mesh=_mesh,
    scratch_types=[
        pltpu.VMEM((_P, 2 * _S), jnp.int32),
        [pltpu.VMEM((2 * _S, _D), jnp.float32) for _ in range(_NBUF)],
        [pltpu.SemaphoreType.DMA for _ in range(_NBUF)],
        [pltpu.SemaphoreType.DMA for _ in range(_NBUF)],
    ],
    compiler_params=pltpu.CompilerParams(use_tc_tiling_on_sc=False),
)
def _gather_kernel(idx_hbm, table_hbm, out_hbm, idx_v, rows, gsem, wsem):
    wid = lax.axis_index("s") * 2 + lax.axis_index("c")
    base = wid * _RB
    pltpu.sync_copy(idx_hbm.at[pl.ds(wid * _P, _P)], idx_v)


    def gather_start(i, b):
        pltpu.async_copy(table_hbm.at[idx_v.at[i]], rows[b], gsem[b])

    def gather_wait(i, b):
        pltpu.make_async_copy(table_hbm.at[idx_v.at[i]], rows[b], gsem[b]).wait()

    def write_start(i, b):
        for h in range(2):
            pltpu.async_copy(
                rows[b].at[pl.ds(h * _S, _S)], out_hbm.at[base + 2 * i + h],
                wsem[b],
            )

    def write_wait(i, b):
        for h in range(2):
            pltpu.make_async_copy(
                rows[b].at[pl.ds(h * _S, _S)], out_hbm.at[base + 2 * i + h],
                wsem[b],
            ).wait()

    for b in range(_NBUF):
        gather_start(b, b)

    def group(g, carry):
        i0 = g * _NBUF
        for b in range(_NBUF):
            gather_wait(i0 + b, b)
            write_start(i0 + b, b)
        @pl.when(g + 1 < _NGROUP)
        def _():
            for b in range(_NBUF):
                write_wait(i0 + b, b)
                gather_start(i0 + _NBUF + b, b)

        return carry

    lax.fori_loop(0, _NGROUP, group, 0)
    for b in range(_NBUF):
        write_wait(_P - _NBUF + b, b)


def kernel(input, embeddings):
    idx2 = input.astype(jnp.int32).reshape(_NB // 2, 2 * _S)
    return _gather_kernel(idx2, embeddings)


# speedup vs baseline: 1.1438x; 1.0169x over previous
import functools

import jax
import jax.numpy as jnp
from jax import lax
from jax.experimental import pallas as pl
from jax.experimental.pallas import tpu as pltpu
from jax.experimental.pallas import tpu_sc as plsc

_D = 32
_NB = 16384
_S = 50
_NW = 32
_RB = _NB // _NW
_P = _RB // 2

_NBUF = 8
_NGROUP = _P // _NBUF


_mesh = plsc.VectorSubcoreMesh(core_axis_name="c", subcore_axis_name="s")


@functools.partial(
    pl.kernel,
    out_type=jax.ShapeDtypeStruct((_NB, _S, _D), jnp.float32),
    mesh=_mesh,
    scratch_types=[
        pltpu.VMEM((_P, 2 * _S), jnp.int32),
        [pltpu.VMEM((2 * _S, _D), jnp.float32) for _ in range(_NBUF)],
        [pltpu.SemaphoreType.DMA for _ in range(_NBUF)],
        [pltpu.SemaphoreType.DMA for _ in range(_NBUF)],
    ],
    compiler_params=pltpu.CompilerParams(use_tc_tiling_on_sc=False),
)
def _gather_kernel(idx_hbm, table_hbm, out_hbm, idx_v, rows, gsem, wsem):
    wid = lax.axis_index("s") * 2 + lax.axis_index("c")
    base = wid * _RB
    pltpu.sync_copy(idx_hbm.at[pl.ds(wid * _P, _P)], idx_v)


    def gather_start(i, b):
        pltpu.async_copy(table_hbm.at[idx_v.at[i]], rows[b], gsem[b])

    def gather_wait(i, b):
        pltpu.make_async_copy(table_hbm.at[idx_v.at[i]], rows[b], gsem[b]).wait()

    def write_start(i, b):
        for h in range(2):
            pltpu.async_copy(
                rows[b].at[pl.ds(h * _S, _S)], out_hbm.at[base + 2 * i + h],
                wsem[b],
            )

    def write_wait(i, b):
        for h in range(2):
            pltpu.make_async_copy(
                rows[b].at[pl.ds(h * _S, _S)], out_hbm.at[base + 2 * i + h],
                wsem[b],
            ).wait()

    for b in range(_NBUF):
        gather_start(b, b)

    def group(g, carry):
        i0 = g * _NBUF
        for b in range(_NBUF):
            gather_wait(i0 + b, b)
            write_start(i0 + b, b)
        @pl.when(g + 1 < _NGROUP)
        def _():
            for b in range(_NBUF):
                write_wait(i0 + b, b)
                gather_start(i0 + _NBUF + b, b)

        return carry

    lax.fori_loop(0, _NGROUP, group, 0)
    for b in range(_NBUF):
        write_wait(_P - _NBUF + b, b)


def kernel(input, embeddings):
    idx2 = input.astype(jnp.int32).reshape(_NB // 2, 2 * _S)
    return _gather_kernel(idx2, embeddings)
